# Initial kernel scaffold; baseline (speedup 1.0000x reference)
import jax
import jax.numpy as jnp
from jax.experimental import pallas as pl


def kernel(input, weight):
    raise NotImplementedError("write your pallas kernel here")


# speedup vs baseline: 1.8422x; 1.8422x over previous
import functools

import jax
import jax.numpy as jnp
from jax import lax
from jax.experimental import pallas as pl
from jax.experimental.pallas import tpu as pltpu
from jax.experimental.pallas import tpu_sc as plsc


def _emb_call(N, D, n_per_w, C):
    n_chunks = n_per_w // C
    mesh = plsc.VectorSubcoreMesh(core_axis_name="c", subcore_axis_name="s")

    @functools.partial(
        pl.kernel,
        mesh=mesh,
        out_type=jax.ShapeDtypeStruct((N, D), jnp.float32),
        scratch_types=[
            pltpu.VMEM((n_per_w,), jnp.int32),
            pltpu.VMEM((C, D), jnp.float32),
            pltpu.SemaphoreType.DMA,
        ],
        compiler_params=pltpu.CompilerParams(use_tc_tiling_on_sc=False),
    )
    def _emb(idx_hbm, table_hbm, out_hbm, idx_v, rows_v, sem):
        wid = lax.axis_index("s") * 2 + lax.axis_index("c")
        base = wid * n_per_w
        pltpu.sync_copy(idx_hbm.at[pl.ds(base, n_per_w)], idx_v)

        def body(g, carry):
            off = g * C
            pltpu.async_copy(
                table_hbm.at[idx_v.at[pl.ds(off, C)]], rows_v, sem
            ).wait()
            pltpu.sync_copy(rows_v, out_hbm.at[pl.ds(base + off, C)])
            return carry

        lax.fori_loop(0, n_chunks, body, 0)

    return _emb


def kernel(input, weight):
    B, H = input.shape
    V, D = weight.shape
    N = B * H
    NW = 32
    n_per_w = N // NW
    C = 640
    idx_flat = input.reshape(N)
    out = _emb_call(N, D, n_per_w, C)(idx_flat, weight)
    return out.reshape(B, H, D)


# speedup vs baseline: 1.8732x; 1.0168x over previous
import functools

import jax
import jax.numpy as jnp
from jax import lax
from jax.experimental import pallas as pl
from jax.experimental.pallas import tpu as pltpu
from jax.experimental.pallas import tpu_sc as plsc


def _emb_call(N, D, n_per_w, C):
    n_chunks = n_per_w // C
    n_pairs = n_chunks // 2
    mesh = plsc.VectorSubcoreMesh(core_axis_name="c", subcore_axis_name="s")

    @functools.partial(
        pl.kernel,
        mesh=mesh,
        out_type=jax.ShapeDtypeStruct((N, D), jnp.float32),
        scratch_types=[
            pltpu.VMEM((n_per_w,), jnp.int32),
            pltpu.VMEM((C, D), jnp.float32),
            pltpu.VMEM((C, D), jnp.float32),
            pltpu.SemaphoreType.DMA,
            pltpu.SemaphoreType.DMA,
            pltpu.SemaphoreType.DMA,
            pltpu.SemaphoreType.DMA,
        ],
        compiler_params=pltpu.CompilerParams(use_tc_tiling_on_sc=False),
    )
    def _emb(idx_hbm, table_hbm, out_hbm, idx_v, rows0, rows1,
             gsem0, gsem1, osem0, osem1):
        wid = lax.axis_index("s") * 2 + lax.axis_index("c")
        base = wid * n_per_w
        pltpu.sync_copy(idx_hbm.at[pl.ds(base, n_per_w)], idx_v)

        def gdesc(g, rows, sem):
            return pltpu.make_async_copy(
                table_hbm.at[idx_v.at[pl.ds(g * C, C)]], rows, sem)

        def odesc(g, rows, sem):
            return pltpu.make_async_copy(
                rows, out_hbm.at[pl.ds(base + g * C, C)], sem)

        gdesc(0, rows0, gsem0).start()

        def pair(i, carry):
            g0 = 2 * i

            @pl.when(i > 0)
            def _():
                odesc(g0 - 1, rows1, osem1).wait()

            gdesc(g0 + 1, rows1, gsem1).start()
            gdesc(g0, rows0, gsem0).wait()
            odesc(g0, rows0, osem0).start()
            odesc(g0, rows0, osem0).wait()

            @pl.when(g0 + 2 < n_chunks)
            def _():
                gdesc(g0 + 2, rows0, gsem0).start()

            gdesc(g0 + 1, rows1, gsem1).wait()
            odesc(g0 + 1, rows1, osem1).start()
            return carry

        lax.fori_loop(0, n_pairs, pair, 0)
        odesc(n_chunks - 1, rows1, osem1).wait()

    return _emb


def kernel(input, weight):
    B, H = input.shape
    V, D = weight.shape
    N = B * H
    NW = 32
    n_per_w = N // NW
    C = 800
    idx_flat = input.reshape(N)
    out = _emb_call(N, D, n_per_w, C)(idx_flat, weight)
    return out.reshape(B, H, D)
